# baseline (speedup 1.0000x reference)
import jax
import jax.numpy as jnp
from jax.experimental import pallas as pl
from jax.experimental.pallas import tpu as pltpu

NB = 4
DEPTH = 2


def _gate_chunk(xb, w1t, w2t, b1, b2x2, inv_hw):
    Bb = xb.shape[0]
    avg_rows = []
    max_rows = []
    for b in range(Bb):
        avg_rows.append(jnp.sum(xb[b], axis=0, keepdims=True) * inv_hw)
        max_rows.append(jnp.max(xb[b], axis=0, keepdims=True))
    avg_mat = jnp.concatenate(avg_rows, axis=0)
    max_mat = jnp.concatenate(max_rows, axis=0)
    dn = (((1,), (0,)), ((), ()))
    h_a = jax.lax.dot_general(avg_mat, w1t, dn,
                              preferred_element_type=jnp.float32)
    h_m = jax.lax.dot_general(max_mat, w1t, dn,
                              preferred_element_type=jnp.float32)
    h_sum = jnp.maximum(h_a + b1, 0.0) + jnp.maximum(h_m + b1, 0.0)
    att = jax.lax.dot_general(h_sum, w2t, dn,
                              preferred_element_type=jnp.float32)
    scale = jax.nn.sigmoid(att + b2x2)
    outs = [xb[b] * scale[b:b + 1, :] for b in range(Bb)]
    return jnp.stack(outs, axis=0)


def _body(x_hbm, w1t_hbm, w2t_hbm, b1_hbm, b2_hbm, o_hbm,
          xbuf, w1t_s, w2t_s, b1_s, b2_s, lsem, ssem, wsem):
    B, HW, C = x_hbm.shape
    Bb = xbuf.shape[1]
    nch = B // Bb
    inv_hw = 1.0 / HW

    def load(i):
        return pltpu.make_async_copy(
            x_hbm.at[pl.ds(i * Bb, Bb)], xbuf.at[i % NB], lsem.at[i % NB])

    def store(i):
        return pltpu.make_async_copy(
            xbuf.at[i % NB], o_hbm.at[pl.ds(i * Bb, Bb)], ssem.at[i % NB])

    for i in range(min(DEPTH, nch)):
        load(i).start()
    pltpu.make_async_copy(w1t_hbm, w1t_s, wsem.at[0]).start()
    pltpu.make_async_copy(w2t_hbm, w2t_s, wsem.at[1]).start()
    pltpu.make_async_copy(b1_hbm, b1_s, wsem.at[2]).start()
    pltpu.make_async_copy(b2_hbm, b2_s, wsem.at[3]).start()
    pltpu.make_async_copy(w1t_hbm, w1t_s, wsem.at[0]).wait()
    pltpu.make_async_copy(w2t_hbm, w2t_s, wsem.at[1]).wait()
    pltpu.make_async_copy(b1_hbm, b1_s, wsem.at[2]).wait()
    pltpu.make_async_copy(b2_hbm, b2_s, wsem.at[3]).wait()
    w1t = w1t_s[...]
    w2t = w2t_s[...]
    b1 = b1_s[...]
    b2x2 = b2_s[...]

    for i in range(nch):
        s = i % NB
        load(i).wait()
        xbuf[s] = _gate_chunk(xbuf[s], w1t, w2t, b1, b2x2, inv_hw)
        store(i).start()
        nxt = i + DEPTH
        if nxt < nch:
            if nxt - NB >= 0:
                store(nxt - NB).wait()
            load(nxt).start()
    for i in range(max(0, nch - NB), nch):
        store(i).wait()


def kernel(x, w1, b1_row, w2, b2_row):
    B, C, H, W = x.shape
    HW = H * W
    hid = w1.shape[0]
    Bb = 8
    while B % Bb != 0:
        Bb -= 1

    x_nhwc = jnp.transpose(x, (0, 2, 3, 1)).reshape(B, HW, C)
    w1t = w1.T
    w2t = w2.T
    b2x2 = 2.0 * b2_row

    any_spec = pl.BlockSpec(memory_space=pl.ANY)
    buf_bytes = NB * Bb * HW * C * 4
    vmem_limit = int(min(56 << 20, buf_bytes + (8 << 20)))

    out = pl.pallas_call(
        _body,
        out_shape=jax.ShapeDtypeStruct((B, HW, C), x.dtype),
        in_specs=[any_spec] * 5,
        out_specs=any_spec,
        scratch_shapes=[
            pltpu.VMEM((NB, Bb, HW, C), jnp.float32),
            pltpu.VMEM((C, hid), jnp.float32),
            pltpu.VMEM((hid, C), jnp.float32),
            pltpu.VMEM((1, hid), jnp.float32),
            pltpu.VMEM((1, C), jnp.float32),
            pltpu.SemaphoreType.DMA((NB,)),
            pltpu.SemaphoreType.DMA((NB,)),
            pltpu.SemaphoreType.DMA((4,)),
        ],
        compiler_params=pltpu.CompilerParams(
            vmem_limit_bytes=vmem_limit,
        ),
    )(x_nhwc, w1t, w2t, b1_row, b2x2)

    return out.reshape(B, H, W, C).transpose(0, 3, 1, 2)


# speedup vs baseline: 1.1113x; 1.1113x over previous
import jax
import jax.numpy as jnp
from jax.experimental import pallas as pl
from jax.experimental.pallas import tpu as pltpu


def _gate_body(x_ref, w1_hbm, w2t_hbm, b1_hbm, b2_hbm, o_ref,
               w1_s, w2t_s, b1_s, b2_s, wsem):
    Bb, HW, C = x_ref.shape
    inv_hw = 1.0 / HW

    @pl.when(pl.program_id(0) == 0)
    def _load_weights():
        pltpu.make_async_copy(w1_hbm, w1_s, wsem.at[0]).start()
        pltpu.make_async_copy(w2t_hbm, w2t_s, wsem.at[1]).start()
        pltpu.make_async_copy(b1_hbm, b1_s, wsem.at[2]).start()
        pltpu.make_async_copy(b2_hbm, b2_s, wsem.at[3]).start()
        pltpu.make_async_copy(w1_hbm, w1_s, wsem.at[0]).wait()
        pltpu.make_async_copy(w2t_hbm, w2t_s, wsem.at[1]).wait()
        pltpu.make_async_copy(b1_hbm, b1_s, wsem.at[2]).wait()
        pltpu.make_async_copy(b2_hbm, b2_s, wsem.at[3]).wait()

    avg_rows = []
    max_rows = []
    for b in range(Bb):
        xb = x_ref[b]
        avg_rows.append(jnp.sum(xb, axis=0, keepdims=True) * inv_hw)
        max_rows.append(jnp.max(xb, axis=0, keepdims=True))
    avg_mat = jnp.concatenate(avg_rows, axis=0)
    max_mat = jnp.concatenate(max_rows, axis=0)

    dn_tb = (((1,), (1,)), ((), ()))
    dn = (((1,), (0,)), ((), ()))
    h_a = jax.lax.dot_general(avg_mat, w1_s[...], dn_tb,
                              preferred_element_type=jnp.float32)
    h_m = jax.lax.dot_general(max_mat, w1_s[...], dn_tb,
                              preferred_element_type=jnp.float32)
    b1 = b1_s[...]
    h_sum = jnp.maximum(h_a + b1, 0.0) + jnp.maximum(h_m + b1, 0.0)
    att = jax.lax.dot_general(h_sum, w2t_s[...], dn,
                              preferred_element_type=jnp.float32)
    scale = jax.nn.sigmoid(att + 2.0 * b2_s[...])

    for b in range(Bb):
        o_ref[b] = x_ref[b] * scale[b:b + 1, :]


def _pick_bb(batch, per_batch_bytes, target_bytes=8 << 20):
    bb = max(1, min(batch, target_bytes // max(per_batch_bytes, 1)))
    while batch % bb != 0:
        bb -= 1
    return bb


def kernel(x, w1, b1_row, w2, b2_row):
    B, C, H, W = x.shape
    HW = H * W
    hid = w1.shape[0]

    per_batch_bytes = C * HW * x.dtype.itemsize
    Bb = _pick_bb(B, per_batch_bytes)
    steps = B // Bb

    x_nhwc = jnp.transpose(x, (0, 2, 3, 1)).reshape(B, HW, C)
    w2t = w2.T

    feat_spec = pl.BlockSpec((Bb, HW, C), lambda i: (i, 0, 0))
    any_spec = pl.BlockSpec(memory_space=pl.ANY)
    block_bytes = Bb * per_batch_bytes
    vmem_limit = int(min(56 << 20, 4 * block_bytes + (8 << 20)))

    out = pl.pallas_call(
        _gate_body,
        out_shape=jax.ShapeDtypeStruct((B, HW, C), x.dtype),
        grid=(steps,),
        in_specs=[feat_spec, any_spec, any_spec, any_spec, any_spec],
        out_specs=feat_spec,
        scratch_shapes=[
            pltpu.VMEM((hid, C), jnp.float32),
            pltpu.VMEM((hid, C), jnp.float32),
            pltpu.VMEM((1, hid), jnp.float32),
            pltpu.VMEM((1, C), jnp.float32),
            pltpu.SemaphoreType.DMA((4,)),
        ],
        compiler_params=pltpu.CompilerParams(
            dimension_semantics=("arbitrary",),
            vmem_limit_bytes=vmem_limit,
        ),
    )(x_nhwc, w1, w2t, b1_row, b2_row)

    return out.reshape(B, H, W, C).transpose(0, 3, 1, 2)
